# Initial kernel scaffold; baseline (speedup 1.0000x reference)
import jax
import jax.numpy as jnp
from jax.experimental import pallas as pl


def kernel(x, sub):
    raise NotImplementedError("write your pallas kernel here")


# speedup vs baseline: 11.3497x; 11.3497x over previous
import jax
import jax.numpy as jnp
from jax.experimental import pallas as pl
from jax.experimental.pallas import tpu as pltpu

N_SUBS = 16
EPS = 1e-5

BBLK = 128
FBLK = 3968
BBLK2 = 16


def _stats_kernel(sub_ref, x_ref, mean_ref, rstd_ref, sum_acc, sq_acc, cnt_acc):
    nb = pl.num_programs(1)
    b = pl.program_id(1)
    xb = x_ref[...]
    subrow = sub_ref[pl.ds(b, 1), :]
    iota = jax.lax.broadcasted_iota(jnp.int32, (N_SUBS, BBLK), 0)
    onehot = (iota == subrow).astype(jnp.float32)
    psum = jax.lax.dot(onehot, xb, preferred_element_type=jnp.float32)
    psq = jax.lax.dot(onehot, xb * xb, preferred_element_type=jnp.float32)

    @pl.when(b == 0)
    def _():
        sum_acc[...] = psum
        sq_acc[...] = psq
        cnt_acc[...] = onehot

    @pl.when(b > 0)
    def _():
        sum_acc[...] += psum
        sq_acc[...] += psq
        cnt_acc[...] += onehot

    @pl.when(b == nb - 1)
    def _():
        n = jnp.sum(cnt_acc[...], axis=1, keepdims=True)
        nf = jnp.maximum(n, 1.0)
        mean = sum_acc[...] / nf
        var = (sq_acc[...] - n * mean * mean) / jnp.maximum(n - 1.0, 1.0)
        var = jnp.maximum(var, 0.0)
        mean_ref[...] = mean
        rstd_ref[...] = 1.0 / (jnp.sqrt(var) + EPS)


def _norm_kernel(sub_ref, x_ref, mean_ref, rstd_ref, out_ref):
    b = pl.program_id(0)
    w = x_ref.shape[-1]
    inv_wm1 = 1.0 / (w - 1)
    for i in range(BBLK2):
        s = sub_ref[b * BBLK2 + i]
        y = (x_ref[i] - mean_ref[s]) * rstd_ref[s]
        m2 = jnp.mean(y, axis=-1, keepdims=True)
        d = y - m2
        v2 = jnp.sum(d * d, axis=-1, keepdims=True) * inv_wm1
        out_ref[i] = d / (jnp.sqrt(v2) + EPS)


def kernel(x, sub):
    B, C, H, W = x.shape
    F = C * H * W
    R = C * H
    nb = B // BBLK
    nf = F // FBLK

    x2d = x.reshape(B, F)
    sub2d = sub.reshape(nb, BBLK)

    mean2d, rstd2d = pl.pallas_call(
        _stats_kernel,
        grid=(nf, nb),
        in_specs=[
            pl.BlockSpec((nb, BBLK), lambda f, b: (0, 0)),
            pl.BlockSpec((BBLK, FBLK), lambda f, b: (b, f)),
        ],
        out_specs=[
            pl.BlockSpec((N_SUBS, FBLK), lambda f, b: (0, f)),
            pl.BlockSpec((N_SUBS, FBLK), lambda f, b: (0, f)),
        ],
        out_shape=[
            jax.ShapeDtypeStruct((N_SUBS, F), jnp.float32),
            jax.ShapeDtypeStruct((N_SUBS, F), jnp.float32),
        ],
        scratch_shapes=[
            pltpu.VMEM((N_SUBS, FBLK), jnp.float32),
            pltpu.VMEM((N_SUBS, FBLK), jnp.float32),
            pltpu.VMEM((N_SUBS, BBLK), jnp.float32),
        ],
    )(sub2d, x2d)

    x3 = x.reshape(B, R, W)
    mean3 = mean2d.reshape(N_SUBS, R, W)
    rstd3 = rstd2d.reshape(N_SUBS, R, W)

    out3 = pl.pallas_call(
        _norm_kernel,
        grid_spec=pltpu.PrefetchScalarGridSpec(
            num_scalar_prefetch=1,
            grid=(B // BBLK2,),
            in_specs=[
                pl.BlockSpec((BBLK2, R, W), lambda b, sub_ref: (b, 0, 0)),
                pl.BlockSpec((N_SUBS, R, W), lambda b, sub_ref: (0, 0, 0)),
                pl.BlockSpec((N_SUBS, R, W), lambda b, sub_ref: (0, 0, 0)),
            ],
            out_specs=pl.BlockSpec((BBLK2, R, W), lambda b, sub_ref: (b, 0, 0)),
        ),
        out_shape=jax.ShapeDtypeStruct((B, R, W), jnp.float32),
    )(sub, x3, mean3, rstd3)

    return out3.reshape(B, C, H, W)


# speedup vs baseline: 13.5345x; 1.1925x over previous
import jax
import jax.numpy as jnp
from jax.experimental import pallas as pl
from jax.experimental.pallas import tpu as pltpu

N_SUBS = 16
EPS = 1e-5

BBLK = 128
FBLK = 3968
BBLK2 = 16


def _stats_kernel(sub_ref, x_ref, mean_ref, rstd_ref, sum_acc, sq_acc, cnt_acc):
    nb = pl.num_programs(1)
    b = pl.program_id(1)
    xb = x_ref[...]
    subrow = sub_ref[pl.ds(b, 1), :]
    iota = jax.lax.broadcasted_iota(jnp.int32, (N_SUBS, BBLK), 0)
    onehot = (iota == subrow).astype(jnp.float32)
    psum = jax.lax.dot(onehot, xb, preferred_element_type=jnp.float32)
    psq = jax.lax.dot(onehot, xb * xb, preferred_element_type=jnp.float32)

    @pl.when(b == 0)
    def _():
        sum_acc[...] = psum
        sq_acc[...] = psq
        cnt_acc[...] = onehot

    @pl.when(b > 0)
    def _():
        sum_acc[...] += psum
        sq_acc[...] += psq
        cnt_acc[...] += onehot

    @pl.when(b == nb - 1)
    def _():
        n = jnp.sum(cnt_acc[...], axis=1, keepdims=True)
        nf = jnp.maximum(n, 1.0)
        mean = sum_acc[...] / nf
        var = (sq_acc[...] - n * mean * mean) / jnp.maximum(n - 1.0, 1.0)
        var = jnp.maximum(var, 0.0)
        mean_ref[...] = mean
        rstd_ref[...] = 1.0 / (jnp.sqrt(var) + EPS)


def _norm_kernel(sub_ref, x_ref, mean_ref, rstd_ref, out_ref):
    b = pl.program_id(0)
    w = x_ref.shape[-1]
    inv_wm1 = 1.0 / (w - 1)
    for i in range(BBLK2):
        s = sub_ref[b * BBLK2 + i]
        y = (x_ref[i] - mean_ref[s]) * rstd_ref[s]
        m2 = jnp.mean(y, axis=-1, keepdims=True)
        d = y - m2
        v2 = jnp.sum(d * d, axis=-1, keepdims=True) * inv_wm1
        out_ref[i] = d / (jnp.sqrt(v2) + EPS)


def kernel(x, sub):
    B, C, H, W = x.shape
    F = C * H * W
    R = C * H
    nb = B // BBLK
    nf = F // FBLK

    x2d = x.reshape(B, F)
    sub2d = sub.reshape(nb, BBLK)

    mean2d, rstd2d = pl.pallas_call(
        _stats_kernel,
        grid=(nf, nb),
        in_specs=[
            pl.BlockSpec((nb, BBLK), lambda f, b: (0, 0)),
            pl.BlockSpec((BBLK, FBLK), lambda f, b: (b, f)),
        ],
        out_specs=[
            pl.BlockSpec((N_SUBS, FBLK), lambda f, b: (0, f)),
            pl.BlockSpec((N_SUBS, FBLK), lambda f, b: (0, f)),
        ],
        out_shape=[
            jax.ShapeDtypeStruct((N_SUBS, F), jnp.float32),
            jax.ShapeDtypeStruct((N_SUBS, F), jnp.float32),
        ],
        scratch_shapes=[
            pltpu.VMEM((N_SUBS, FBLK), jnp.float32),
            pltpu.VMEM((N_SUBS, FBLK), jnp.float32),
            pltpu.VMEM((N_SUBS, BBLK), jnp.float32),
        ],
    )(sub2d, x2d)

    mean4 = mean2d.reshape(N_SUBS, C, H, W)
    rstd4 = rstd2d.reshape(N_SUBS, C, H, W)

    out = pl.pallas_call(
        _norm_kernel,
        grid_spec=pltpu.PrefetchScalarGridSpec(
            num_scalar_prefetch=1,
            grid=(B // BBLK2,),
            in_specs=[
                pl.BlockSpec((BBLK2, C, H, W), lambda b, sub_ref: (b, 0, 0, 0)),
                pl.BlockSpec((N_SUBS, C, H, W), lambda b, sub_ref: (0, 0, 0, 0)),
                pl.BlockSpec((N_SUBS, C, H, W), lambda b, sub_ref: (0, 0, 0, 0)),
            ],
            out_specs=pl.BlockSpec((BBLK2, C, H, W), lambda b, sub_ref: (b, 0, 0, 0)),
        ),
        out_shape=jax.ShapeDtypeStruct((B, C, H, W), jnp.float32),
    )(sub, x, mean4, rstd4)

    return out


# speedup vs baseline: 20.5477x; 1.5182x over previous
import jax
import jax.numpy as jnp
from jax.experimental import pallas as pl
from jax.experimental.pallas import tpu as pltpu

N_SUBS = 16
EPS = 1e-5

BBLK = 128
FBLK = 3968
BBLK2 = 16


def _stats_kernel(sub_ref, x_ref, mean_ref, rstd_ref, sum_acc, sq_acc, cnt_acc):
    nb = pl.num_programs(1)
    b = pl.program_id(1)
    xb4 = x_ref[...]
    xb = xb4.reshape(xb4.shape[0], -1)
    subrow = sub_ref[pl.ds(b, 1), :]
    iota = jax.lax.broadcasted_iota(jnp.int32, (N_SUBS, BBLK), 0)
    onehot = (iota == subrow).astype(jnp.float32)
    psum = jax.lax.dot(onehot, xb, preferred_element_type=jnp.float32)
    psq = jax.lax.dot(onehot, xb * xb, preferred_element_type=jnp.float32)

    @pl.when(b == 0)
    def _():
        sum_acc[...] = psum
        sq_acc[...] = psq
        cnt_acc[...] = onehot

    @pl.when(b > 0)
    def _():
        sum_acc[...] += psum
        sq_acc[...] += psq
        cnt_acc[...] += onehot

    @pl.when(b == nb - 1)
    def _():
        n = jnp.sum(cnt_acc[...], axis=1, keepdims=True)
        nf = jnp.maximum(n, 1.0)
        mean = sum_acc[...] / nf
        var = (sq_acc[...] - n * mean * mean) / jnp.maximum(n - 1.0, 1.0)
        var = jnp.maximum(var, 0.0)
        mean_ref[...] = mean
        rstd_ref[...] = 1.0 / (jnp.sqrt(var) + EPS)


def _norm_kernel(sub_ref, x_ref, mean_ref, rstd_ref, out_ref):
    b = pl.program_id(0)
    w = x_ref.shape[-1]
    inv_wm1 = 1.0 / (w - 1)
    for i in range(BBLK2):
        s = sub_ref[b * BBLK2 + i]
        y = (x_ref[i] - mean_ref[s]) * rstd_ref[s]
        m2 = jnp.mean(y, axis=-1, keepdims=True)
        d = y - m2
        v2 = jnp.sum(d * d, axis=-1, keepdims=True) * inv_wm1
        out_ref[i] = d / (jnp.sqrt(v2) + EPS)


def kernel(x, sub):
    B, C, H, W = x.shape
    HW = H * W
    nb = B // BBLK

    sub2d = sub.reshape(nb, BBLK)

    mean2d, rstd2d = pl.pallas_call(
        _stats_kernel,
        grid=(C, nb),
        in_specs=[
            pl.BlockSpec((nb, BBLK), lambda c, b: (0, 0)),
            pl.BlockSpec((BBLK, 1, H, W), lambda c, b: (b, c, 0, 0)),
        ],
        out_specs=[
            pl.BlockSpec((N_SUBS, HW), lambda c, b: (0, c)),
            pl.BlockSpec((N_SUBS, HW), lambda c, b: (0, c)),
        ],
        out_shape=[
            jax.ShapeDtypeStruct((N_SUBS, C * HW), jnp.float32),
            jax.ShapeDtypeStruct((N_SUBS, C * HW), jnp.float32),
        ],
        scratch_shapes=[
            pltpu.VMEM((N_SUBS, HW), jnp.float32),
            pltpu.VMEM((N_SUBS, HW), jnp.float32),
            pltpu.VMEM((N_SUBS, BBLK), jnp.float32),
        ],
    )(sub2d, x)

    mean4 = mean2d.reshape(N_SUBS, C, H, W)
    rstd4 = rstd2d.reshape(N_SUBS, C, H, W)

    out = pl.pallas_call(
        _norm_kernel,
        grid_spec=pltpu.PrefetchScalarGridSpec(
            num_scalar_prefetch=1,
            grid=(B // BBLK2,),
            in_specs=[
                pl.BlockSpec((BBLK2, C, H, W), lambda b, sub_ref: (b, 0, 0, 0)),
                pl.BlockSpec((N_SUBS, C, H, W), lambda b, sub_ref: (0, 0, 0, 0)),
                pl.BlockSpec((N_SUBS, C, H, W), lambda b, sub_ref: (0, 0, 0, 0)),
            ],
            out_specs=pl.BlockSpec((BBLK2, C, H, W), lambda b, sub_ref: (b, 0, 0, 0)),
        ),
        out_shape=jax.ShapeDtypeStruct((B, C, H, W), jnp.float32),
    )(sub, x, mean4, rstd4)

    return out


# speedup vs baseline: 21.3110x; 1.0371x over previous
import jax
import jax.numpy as jnp
from jax.experimental import pallas as pl
from jax.experimental.pallas import tpu as pltpu

N_SUBS = 16
EPS = 1e-5

BBLK = 128
FBLK = 3968
BBLK2 = 32


def _stats_kernel(sub_ref, x_ref, mean_ref, rstd_ref, sum_acc, sq_acc, cnt_acc):
    nb = pl.num_programs(1)
    b = pl.program_id(1)
    xb4 = x_ref[...]
    xb = xb4.reshape(xb4.shape[0], -1)
    subrow = sub_ref[pl.ds(b, 1), :]
    iota = jax.lax.broadcasted_iota(jnp.int32, (N_SUBS, BBLK), 0)
    onehot = (iota == subrow).astype(jnp.float32)
    psum = jax.lax.dot(onehot, xb, preferred_element_type=jnp.float32)
    psq = jax.lax.dot(onehot, xb * xb, preferred_element_type=jnp.float32)

    @pl.when(b == 0)
    def _():
        sum_acc[...] = psum
        sq_acc[...] = psq
        cnt_acc[...] = onehot

    @pl.when(b > 0)
    def _():
        sum_acc[...] += psum
        sq_acc[...] += psq
        cnt_acc[...] += onehot

    @pl.when(b == nb - 1)
    def _():
        n = jnp.sum(cnt_acc[...], axis=1, keepdims=True)
        nf = jnp.maximum(n, 1.0)
        mean = sum_acc[...] / nf
        var = (sq_acc[...] - n * mean * mean) / jnp.maximum(n - 1.0, 1.0)
        var = jnp.maximum(var, 0.0)
        mean_ref[...] = mean
        rstd_ref[...] = 1.0 / (jnp.sqrt(var) + EPS)


def _norm_kernel(sub_ref, x_ref, mean_ref, rstd_ref, out_ref):
    b = pl.program_id(0)
    w = x_ref.shape[-1]
    inv_wm1 = 1.0 / (w - 1)
    for i in range(BBLK2):
        s = sub_ref[b * BBLK2 + i]
        y = (x_ref[i] - mean_ref[s]) * rstd_ref[s]
        m2 = jnp.mean(y, axis=-1, keepdims=True)
        d = y - m2
        v2 = jnp.sum(d * d, axis=-1, keepdims=True) * inv_wm1
        out_ref[i] = d / (jnp.sqrt(v2) + EPS)


def kernel(x, sub):
    B, C, H, W = x.shape
    HW = H * W
    nb = B // BBLK

    sub2d = sub.reshape(nb, BBLK)

    mean2d, rstd2d = pl.pallas_call(
        _stats_kernel,
        grid=(C, nb),
        in_specs=[
            pl.BlockSpec((nb, BBLK), lambda c, b: (0, 0)),
            pl.BlockSpec((BBLK, 1, H, W), lambda c, b: (b, c, 0, 0)),
        ],
        out_specs=[
            pl.BlockSpec((N_SUBS, HW), lambda c, b: (0, c)),
            pl.BlockSpec((N_SUBS, HW), lambda c, b: (0, c)),
        ],
        out_shape=[
            jax.ShapeDtypeStruct((N_SUBS, C * HW), jnp.float32),
            jax.ShapeDtypeStruct((N_SUBS, C * HW), jnp.float32),
        ],
        scratch_shapes=[
            pltpu.VMEM((N_SUBS, HW), jnp.float32),
            pltpu.VMEM((N_SUBS, HW), jnp.float32),
            pltpu.VMEM((N_SUBS, BBLK), jnp.float32),
        ],
    )(sub2d, x)

    mean4 = mean2d.reshape(N_SUBS, C, H, W)
    rstd4 = rstd2d.reshape(N_SUBS, C, H, W)

    out = pl.pallas_call(
        _norm_kernel,
        grid_spec=pltpu.PrefetchScalarGridSpec(
            num_scalar_prefetch=1,
            grid=(B // BBLK2,),
            in_specs=[
                pl.BlockSpec((BBLK2, C, H, W), lambda b, sub_ref: (b, 0, 0, 0)),
                pl.BlockSpec((N_SUBS, C, H, W), lambda b, sub_ref: (0, 0, 0, 0)),
                pl.BlockSpec((N_SUBS, C, H, W), lambda b, sub_ref: (0, 0, 0, 0)),
            ],
            out_specs=pl.BlockSpec((BBLK2, C, H, W), lambda b, sub_ref: (b, 0, 0, 0)),
        ),
        out_shape=jax.ShapeDtypeStruct((B, C, H, W), jnp.float32),
    )(sub, x, mean4, rstd4)

    return out


# speedup vs baseline: 21.3832x; 1.0034x over previous
import jax
import jax.numpy as jnp
from jax.experimental import pallas as pl
from jax.experimental.pallas import tpu as pltpu

N_SUBS = 16
EPS = 1e-5

BBLK = 128
FBLK = 3968
BBLK2 = 64


def _stats_kernel(sub_ref, x_ref, mean_ref, rstd_ref, sum_acc, sq_acc, cnt_acc):
    nb = pl.num_programs(1)
    b = pl.program_id(1)
    xb4 = x_ref[...]
    xb = xb4.reshape(xb4.shape[0], -1)
    subrow = sub_ref[pl.ds(b, 1), :]
    iota = jax.lax.broadcasted_iota(jnp.int32, (N_SUBS, BBLK), 0)
    onehot = (iota == subrow).astype(jnp.float32)
    psum = jax.lax.dot(onehot, xb, preferred_element_type=jnp.float32)
    psq = jax.lax.dot(onehot, xb * xb, preferred_element_type=jnp.float32)

    @pl.when(b == 0)
    def _():
        sum_acc[...] = psum
        sq_acc[...] = psq
        cnt_acc[...] = onehot

    @pl.when(b > 0)
    def _():
        sum_acc[...] += psum
        sq_acc[...] += psq
        cnt_acc[...] += onehot

    @pl.when(b == nb - 1)
    def _():
        n = jnp.sum(cnt_acc[...], axis=1, keepdims=True)
        nf = jnp.maximum(n, 1.0)
        mean = sum_acc[...] / nf
        var = (sq_acc[...] - n * mean * mean) / jnp.maximum(n - 1.0, 1.0)
        var = jnp.maximum(var, 0.0)
        mean_ref[...] = mean
        rstd_ref[...] = 1.0 / (jnp.sqrt(var) + EPS)


def _norm_kernel(sub_ref, x_ref, mean_ref, rstd_ref, out_ref):
    b = pl.program_id(0)
    w = x_ref.shape[-1]
    inv_wm1 = 1.0 / (w - 1)
    for i in range(BBLK2):
        s = sub_ref[b * BBLK2 + i]
        y = (x_ref[i] - mean_ref[s]) * rstd_ref[s]
        m2 = jnp.mean(y, axis=-1, keepdims=True)
        d = y - m2
        v2 = jnp.sum(d * d, axis=-1, keepdims=True) * inv_wm1
        out_ref[i] = d / (jnp.sqrt(v2) + EPS)


def kernel(x, sub):
    B, C, H, W = x.shape
    HW = H * W
    nb = B // BBLK

    sub2d = sub.reshape(nb, BBLK)

    mean2d, rstd2d = pl.pallas_call(
        _stats_kernel,
        grid=(C, nb),
        in_specs=[
            pl.BlockSpec((nb, BBLK), lambda c, b: (0, 0)),
            pl.BlockSpec((BBLK, 1, H, W), lambda c, b: (b, c, 0, 0)),
        ],
        out_specs=[
            pl.BlockSpec((N_SUBS, HW), lambda c, b: (0, c)),
            pl.BlockSpec((N_SUBS, HW), lambda c, b: (0, c)),
        ],
        out_shape=[
            jax.ShapeDtypeStruct((N_SUBS, C * HW), jnp.float32),
            jax.ShapeDtypeStruct((N_SUBS, C * HW), jnp.float32),
        ],
        scratch_shapes=[
            pltpu.VMEM((N_SUBS, HW), jnp.float32),
            pltpu.VMEM((N_SUBS, HW), jnp.float32),
            pltpu.VMEM((N_SUBS, BBLK), jnp.float32),
        ],
    )(sub2d, x)

    mean4 = mean2d.reshape(N_SUBS, C, H, W)
    rstd4 = rstd2d.reshape(N_SUBS, C, H, W)

    out = pl.pallas_call(
        _norm_kernel,
        grid_spec=pltpu.PrefetchScalarGridSpec(
            num_scalar_prefetch=1,
            grid=(B // BBLK2,),
            in_specs=[
                pl.BlockSpec((BBLK2, C, H, W), lambda b, sub_ref: (b, 0, 0, 0)),
                pl.BlockSpec((N_SUBS, C, H, W), lambda b, sub_ref: (0, 0, 0, 0)),
                pl.BlockSpec((N_SUBS, C, H, W), lambda b, sub_ref: (0, 0, 0, 0)),
            ],
            out_specs=pl.BlockSpec((BBLK2, C, H, W), lambda b, sub_ref: (b, 0, 0, 0)),
        ),
        out_shape=jax.ShapeDtypeStruct((B, C, H, W), jnp.float32),
    )(sub, x, mean4, rstd4)

    return out


# speedup vs baseline: 21.7417x; 1.0168x over previous
import jax
import jax.numpy as jnp
from jax.experimental import pallas as pl
from jax.experimental.pallas import tpu as pltpu

N_SUBS = 16
EPS = 1e-5

BBLK = 128
FBLK = 3968
BBLK2 = 64


def _stats_kernel(sub_ref, x_ref, mean_ref, rstd_ref, sum_acc, sq_acc, cnt_acc):
    nb = pl.num_programs(1)
    b = pl.program_id(1)
    xb4 = x_ref[...]
    xb = xb4.reshape(xb4.shape[0], -1)
    subrow = sub_ref[pl.ds(b, 1), :]
    iota = jax.lax.broadcasted_iota(jnp.int32, (N_SUBS, BBLK), 0)
    onehot = (iota == subrow).astype(jnp.float32)
    psum = jax.lax.dot(onehot, xb, preferred_element_type=jnp.float32)
    psq = jax.lax.dot(onehot, xb * xb, preferred_element_type=jnp.float32)

    @pl.when(b == 0)
    def _():
        sum_acc[...] = psum
        sq_acc[...] = psq
        cnt_acc[...] = onehot

    @pl.when(b > 0)
    def _():
        sum_acc[...] += psum
        sq_acc[...] += psq
        cnt_acc[...] += onehot

    @pl.when(b == nb - 1)
    def _():
        n = jnp.sum(cnt_acc[...], axis=1, keepdims=True)
        nf = jnp.maximum(n, 1.0)
        mean = sum_acc[...] / nf
        var = (sq_acc[...] - n * mean * mean) / jnp.maximum(n - 1.0, 1.0)
        var = jnp.maximum(var, 0.0)
        rstd = 1.0 / (jnp.sqrt(var) + EPS)
        hw = mean_ref.shape
        mean_ref[...] = mean.reshape(hw)
        rstd_ref[...] = rstd.reshape(hw)


def _norm_kernel(sub_ref, x_ref, mean_ref, rstd_ref, out_ref):
    b = pl.program_id(0)
    w = x_ref.shape[-1]
    inv_wm1 = 1.0 / (w - 1)
    for i in range(BBLK2):
        s = sub_ref[b * BBLK2 + i]
        y = (x_ref[i] - mean_ref[s]) * rstd_ref[s]
        m2 = jnp.mean(y, axis=-1, keepdims=True)
        d = y - m2
        v2 = jnp.sum(d * d, axis=-1, keepdims=True) * inv_wm1
        out_ref[i] = d / (jnp.sqrt(v2) + EPS)


def kernel(x, sub):
    B, C, H, W = x.shape
    HW = H * W
    nb = B // BBLK

    sub2d = sub.reshape(nb, BBLK)

    mean4, rstd4 = pl.pallas_call(
        _stats_kernel,
        grid=(C, nb),
        in_specs=[
            pl.BlockSpec((nb, BBLK), lambda c, b: (0, 0)),
            pl.BlockSpec((BBLK, 1, H, W), lambda c, b: (b, c, 0, 0)),
        ],
        out_specs=[
            pl.BlockSpec((N_SUBS, 1, H, W), lambda c, b: (0, c, 0, 0)),
            pl.BlockSpec((N_SUBS, 1, H, W), lambda c, b: (0, c, 0, 0)),
        ],
        out_shape=[
            jax.ShapeDtypeStruct((N_SUBS, C, H, W), jnp.float32),
            jax.ShapeDtypeStruct((N_SUBS, C, H, W), jnp.float32),
        ],
        scratch_shapes=[
            pltpu.VMEM((N_SUBS, HW), jnp.float32),
            pltpu.VMEM((N_SUBS, HW), jnp.float32),
            pltpu.VMEM((N_SUBS, BBLK), jnp.float32),
        ],
    )(sub2d, x)

    out = pl.pallas_call(
        _norm_kernel,
        grid_spec=pltpu.PrefetchScalarGridSpec(
            num_scalar_prefetch=1,
            grid=(B // BBLK2,),
            in_specs=[
                pl.BlockSpec((BBLK2, C, H, W), lambda b, sub_ref: (b, 0, 0, 0)),
                pl.BlockSpec((N_SUBS, C, H, W), lambda b, sub_ref: (0, 0, 0, 0)),
                pl.BlockSpec((N_SUBS, C, H, W), lambda b, sub_ref: (0, 0, 0, 0)),
            ],
            out_specs=pl.BlockSpec((BBLK2, C, H, W), lambda b, sub_ref: (b, 0, 0, 0)),
        ),
        out_shape=jax.ShapeDtypeStruct((B, C, H, W), jnp.float32),
    )(sub, x, mean4, rstd4)

    return out


# speedup vs baseline: 22.1100x; 1.0169x over previous
import jax
import jax.numpy as jnp
from jax.experimental import pallas as pl
from jax.experimental.pallas import tpu as pltpu

N_SUBS = 16
EPS = 1e-5

BBLK = 128
BBLK2 = 64


def _stats_kernel(sub_ref, x_ref, mean_ref, rstd_ref, sum_acc, sq_acc, cnt_acc):
    nb = pl.num_programs(0)
    b = pl.program_id(0)
    C = x_ref.shape[1]
    hw = x_ref.shape[2] * x_ref.shape[3]
    subrow = sub_ref[pl.ds(b, 1), :]
    iota = jax.lax.broadcasted_iota(jnp.int32, (N_SUBS, BBLK), 0)
    onehot = (iota == subrow).astype(jnp.float32)
    for c in range(C):
        xc = x_ref[:, c].reshape(BBLK, hw)
        psum = jax.lax.dot(onehot, xc, preferred_element_type=jnp.float32)
        psq = jax.lax.dot(onehot, xc * xc, preferred_element_type=jnp.float32)
        cols = pl.ds(c * hw, hw)

        @pl.when(b == 0)
        def _():
            sum_acc[:, cols] = psum
            sq_acc[:, cols] = psq

        @pl.when(b > 0)
        def _():
            sum_acc[:, cols] += psum
            sq_acc[:, cols] += psq

    @pl.when(b == 0)
    def _():
        cnt_acc[...] = onehot

    @pl.when(b > 0)
    def _():
        cnt_acc[...] += onehot

    @pl.when(b == nb - 1)
    def _():
        n = jnp.sum(cnt_acc[...], axis=1, keepdims=True)
        nf = jnp.maximum(n, 1.0)
        mean = sum_acc[...] / nf
        var = (sq_acc[...] - n * mean * mean) / jnp.maximum(n - 1.0, 1.0)
        var = jnp.maximum(var, 0.0)
        rstd = 1.0 / (jnp.sqrt(var) + EPS)
        shp = mean_ref.shape
        mean_ref[...] = mean.reshape(shp)
        rstd_ref[...] = rstd.reshape(shp)


def _norm_kernel(sub_ref, x_ref, mean_ref, rstd_ref, out_ref):
    b = pl.program_id(0)
    w = x_ref.shape[-1]
    inv_wm1 = 1.0 / (w - 1)
    for i in range(BBLK2):
        s = sub_ref[b * BBLK2 + i]
        y = (x_ref[i] - mean_ref[s]) * rstd_ref[s]
        m2 = jnp.mean(y, axis=-1, keepdims=True)
        d = y - m2
        v2 = jnp.sum(d * d, axis=-1, keepdims=True) * inv_wm1
        out_ref[i] = d / (jnp.sqrt(v2) + EPS)


def kernel(x, sub):
    B, C, H, W = x.shape
    F = C * H * W
    nb = B // BBLK

    sub2d = sub.reshape(nb, BBLK)

    mean4, rstd4 = pl.pallas_call(
        _stats_kernel,
        grid=(nb,),
        in_specs=[
            pl.BlockSpec((nb, BBLK), lambda b: (0, 0)),
            pl.BlockSpec((BBLK, C, H, W), lambda b: (b, 0, 0, 0)),
        ],
        out_specs=[
            pl.BlockSpec((N_SUBS, C, H, W), lambda b: (0, 0, 0, 0)),
            pl.BlockSpec((N_SUBS, C, H, W), lambda b: (0, 0, 0, 0)),
        ],
        out_shape=[
            jax.ShapeDtypeStruct((N_SUBS, C, H, W), jnp.float32),
            jax.ShapeDtypeStruct((N_SUBS, C, H, W), jnp.float32),
        ],
        scratch_shapes=[
            pltpu.VMEM((N_SUBS, F), jnp.float32),
            pltpu.VMEM((N_SUBS, F), jnp.float32),
            pltpu.VMEM((N_SUBS, BBLK), jnp.float32),
        ],
    )(sub2d, x)

    out = pl.pallas_call(
        _norm_kernel,
        grid_spec=pltpu.PrefetchScalarGridSpec(
            num_scalar_prefetch=1,
            grid=(B // BBLK2,),
            in_specs=[
                pl.BlockSpec((BBLK2, C, H, W), lambda b, sub_ref: (b, 0, 0, 0)),
                pl.BlockSpec((N_SUBS, C, H, W), lambda b, sub_ref: (0, 0, 0, 0)),
                pl.BlockSpec((N_SUBS, C, H, W), lambda b, sub_ref: (0, 0, 0, 0)),
            ],
            out_specs=pl.BlockSpec((BBLK2, C, H, W), lambda b, sub_ref: (b, 0, 0, 0)),
        ),
        out_shape=jax.ShapeDtypeStruct((B, C, H, W), jnp.float32),
    )(sub, x, mean4, rstd4)

    return out


# speedup vs baseline: 42.9362x; 1.9419x over previous
import jax
import jax.numpy as jnp
from jax.experimental import pallas as pl
from jax.experimental.pallas import tpu as pltpu

N_SUBS = 16
EPS = 1e-5

BBLK = 128
RCHUNK = 8
BBLK2 = 64


def _stats_kernel(sub_ref, x_ref, mean_ref, rstd_ref, sum_acc, sq_acc, cnt_acc):
    nb = pl.num_programs(0)
    b = pl.program_id(0)
    R = x_ref.shape[1]
    w = x_ref.shape[2]
    subrow = sub_ref[pl.ds(b, 1), :]
    iota = jax.lax.broadcasted_iota(jnp.int32, (N_SUBS, BBLK), 0)
    onehot = (iota == subrow).astype(jnp.float32)
    for k in range(R // RCHUNK):
        xc = x_ref[:, pl.ds(k * RCHUNK, RCHUNK), :].reshape(BBLK, RCHUNK * w)
        psum = jax.lax.dot(onehot, xc, preferred_element_type=jnp.float32)
        psq = jax.lax.dot(onehot, xc * xc, preferred_element_type=jnp.float32)
        cols = pl.ds(k * RCHUNK * w, RCHUNK * w)

        @pl.when(b == 0)
        def _():
            sum_acc[:, cols] = psum
            sq_acc[:, cols] = psq

        @pl.when(b > 0)
        def _():
            sum_acc[:, cols] += psum
            sq_acc[:, cols] += psq

    @pl.when(b == 0)
    def _():
        cnt_acc[...] = onehot

    @pl.when(b > 0)
    def _():
        cnt_acc[...] += onehot

    @pl.when(b == nb - 1)
    def _():
        n = jnp.sum(cnt_acc[...], axis=1, keepdims=True)
        nf = jnp.maximum(n, 1.0)
        mean = sum_acc[...] / nf
        var = (sq_acc[...] - n * mean * mean) / jnp.maximum(n - 1.0, 1.0)
        var = jnp.maximum(var, 0.0)
        rstd = 1.0 / (jnp.sqrt(var) + EPS)
        shp = mean_ref.shape
        mean_ref[...] = mean.reshape(shp)
        rstd_ref[...] = rstd.reshape(shp)


def _norm_kernel(sub_ref, x_ref, mean_ref, rstd_ref, out_ref):
    b = pl.program_id(0)
    w = x_ref.shape[-1]
    inv_wm1 = 1.0 / (w - 1)
    for i in range(BBLK2):
        s = sub_ref[b * BBLK2 + i]
        y = (x_ref[i] - mean_ref[s]) * rstd_ref[s]
        m2 = jnp.mean(y, axis=-1, keepdims=True)
        d = y - m2
        v2 = jnp.sum(d * d, axis=-1, keepdims=True) * inv_wm1
        out_ref[i] = d / (jnp.sqrt(v2) + EPS)


def kernel(x, sub):
    B, C, H, W = x.shape
    R = H * C
    F = R * W
    nb = B // BBLK

    xt = jnp.transpose(x, (0, 2, 1, 3)).reshape(B, R, W)
    sub2d = sub.reshape(nb, BBLK)

    mean3, rstd3 = pl.pallas_call(
        _stats_kernel,
        grid=(nb,),
        in_specs=[
            pl.BlockSpec((nb, BBLK), lambda b: (0, 0)),
            pl.BlockSpec((BBLK, R, W), lambda b: (b, 0, 0)),
        ],
        out_specs=[
            pl.BlockSpec((N_SUBS, R, W), lambda b: (0, 0, 0)),
            pl.BlockSpec((N_SUBS, R, W), lambda b: (0, 0, 0)),
        ],
        out_shape=[
            jax.ShapeDtypeStruct((N_SUBS, R, W), jnp.float32),
            jax.ShapeDtypeStruct((N_SUBS, R, W), jnp.float32),
        ],
        scratch_shapes=[
            pltpu.VMEM((N_SUBS, F), jnp.float32),
            pltpu.VMEM((N_SUBS, F), jnp.float32),
            pltpu.VMEM((N_SUBS, BBLK), jnp.float32),
        ],
    )(sub2d, xt)

    out_t = pl.pallas_call(
        _norm_kernel,
        grid_spec=pltpu.PrefetchScalarGridSpec(
            num_scalar_prefetch=1,
            grid=(B // BBLK2,),
            in_specs=[
                pl.BlockSpec((BBLK2, R, W), lambda b, sub_ref: (b, 0, 0)),
                pl.BlockSpec((N_SUBS, R, W), lambda b, sub_ref: (0, 0, 0)),
                pl.BlockSpec((N_SUBS, R, W), lambda b, sub_ref: (0, 0, 0)),
            ],
            out_specs=pl.BlockSpec((BBLK2, R, W), lambda b, sub_ref: (b, 0, 0)),
        ),
        out_shape=jax.ShapeDtypeStruct((B, R, W), jnp.float32),
    )(sub, xt, mean3, rstd3)

    return jnp.transpose(out_t.reshape(B, H, C, W), (0, 2, 1, 3))


# speedup vs baseline: 46.5006x; 1.0830x over previous
import jax
import jax.numpy as jnp
from jax.experimental import pallas as pl
from jax.experimental.pallas import tpu as pltpu

N_SUBS = 16
EPS = 1e-5

BBLK = 64
RCHUNK = 248
BBLK2 = 64


def _stats_kernel(sub_ref, x_ref, mean_ref, rstd_ref, sum_acc, sq_acc, cnt_acc):
    nb = pl.num_programs(0)
    b = pl.program_id(0)
    R = x_ref.shape[1]
    w = x_ref.shape[2]
    subrow = sub_ref[pl.ds(b, 1), :]
    iota = jax.lax.broadcasted_iota(jnp.int32, (N_SUBS, BBLK), 0)
    onehot = (iota == subrow).astype(jnp.float32)
    for k in range(R // RCHUNK):
        xc = x_ref[:, pl.ds(k * RCHUNK, RCHUNK), :].reshape(BBLK, RCHUNK * w)
        psum = jax.lax.dot(onehot, xc, preferred_element_type=jnp.float32)
        psq = jax.lax.dot(onehot, xc * xc, preferred_element_type=jnp.float32)
        cols = pl.ds(k * RCHUNK * w, RCHUNK * w)

        @pl.when(b == 0)
        def _():
            sum_acc[:, cols] = psum
            sq_acc[:, cols] = psq

        @pl.when(b > 0)
        def _():
            sum_acc[:, cols] += psum
            sq_acc[:, cols] += psq

    @pl.when(b == 0)
    def _():
        cnt_acc[...] = onehot

    @pl.when(b > 0)
    def _():
        cnt_acc[...] += onehot

    @pl.when(b == nb - 1)
    def _():
        n = jnp.sum(cnt_acc[...], axis=1, keepdims=True)
        nf = jnp.maximum(n, 1.0)
        mean = sum_acc[...] / nf
        var = (sq_acc[...] - n * mean * mean) / jnp.maximum(n - 1.0, 1.0)
        var = jnp.maximum(var, 0.0)
        rstd = 1.0 / (jnp.sqrt(var) + EPS)
        shp = mean_ref.shape
        mean_ref[...] = mean.reshape(shp)
        rstd_ref[...] = rstd.reshape(shp)


def _norm_kernel(sub_ref, x_ref, mean_ref, rstd_ref, out_ref):
    b = pl.program_id(0)
    w = x_ref.shape[-1]
    inv_wm1 = 1.0 / (w - 1)
    for i in range(BBLK2):
        s = sub_ref[b * BBLK2 + i]
        y = (x_ref[i] - mean_ref[s]) * rstd_ref[s]
        m2 = jnp.mean(y, axis=-1, keepdims=True)
        d = y - m2
        v2 = jnp.sum(d * d, axis=-1, keepdims=True) * inv_wm1
        out_ref[i] = d / (jnp.sqrt(v2) + EPS)


def kernel(x, sub):
    B, C, H, W = x.shape
    R = H * C
    F = R * W
    nb = B // BBLK

    xt = jnp.transpose(x, (0, 2, 1, 3)).reshape(B, R, W)
    sub2d = sub.reshape(nb, BBLK)

    mean3, rstd3 = pl.pallas_call(
        _stats_kernel,
        grid=(nb,),
        in_specs=[
            pl.BlockSpec((nb, BBLK), lambda b: (0, 0)),
            pl.BlockSpec((BBLK, R, W), lambda b: (b, 0, 0)),
        ],
        out_specs=[
            pl.BlockSpec((N_SUBS, R, W), lambda b: (0, 0, 0)),
            pl.BlockSpec((N_SUBS, R, W), lambda b: (0, 0, 0)),
        ],
        out_shape=[
            jax.ShapeDtypeStruct((N_SUBS, R, W), jnp.float32),
            jax.ShapeDtypeStruct((N_SUBS, R, W), jnp.float32),
        ],
        scratch_shapes=[
            pltpu.VMEM((N_SUBS, F), jnp.float32),
            pltpu.VMEM((N_SUBS, F), jnp.float32),
            pltpu.VMEM((N_SUBS, BBLK), jnp.float32),
        ],
    )(sub2d, xt)

    out_t = pl.pallas_call(
        _norm_kernel,
        grid_spec=pltpu.PrefetchScalarGridSpec(
            num_scalar_prefetch=1,
            grid=(B // BBLK2,),
            in_specs=[
                pl.BlockSpec((BBLK2, R, W), lambda b, sub_ref: (b, 0, 0)),
                pl.BlockSpec((N_SUBS, R, W), lambda b, sub_ref: (0, 0, 0)),
                pl.BlockSpec((N_SUBS, R, W), lambda b, sub_ref: (0, 0, 0)),
            ],
            out_specs=pl.BlockSpec((BBLK2, R, W), lambda b, sub_ref: (b, 0, 0)),
        ),
        out_shape=jax.ShapeDtypeStruct((B, R, W), jnp.float32),
    )(sub, xt, mean3, rstd3)

    return jnp.transpose(out_t.reshape(B, H, C, W), (0, 2, 1, 3))
